# baseline (speedup 1.0000x reference)
import functools

import jax
import jax.numpy as jnp
from jax import lax
from jax.experimental import pallas as pl
from jax.experimental.pallas import tpu as pltpu
from jax.experimental.pallas import tpu_sc as plsc

D = 16
NC, NS = 2, 16
NW = NC * NS
B_TOTAL = 4096 * 200
B_PER_W = B_TOTAL // NW
CH = 2560
N_CH = B_PER_W // CH

_mesh = plsc.VectorSubcoreMesh(core_axis_name="c", subcore_axis_name="s")


@functools.partial(
    pl.kernel,
    mesh=_mesh,
    out_type=jax.ShapeDtypeStruct((B_TOTAL, D), jnp.float32),
    scratch_types=[
        pltpu.VMEM((CH,), jnp.int32),
        pltpu.VMEM((CH, D), jnp.float32),
        pltpu.SemaphoreType.DMA,
    ],
    compiler_params=pltpu.CompilerParams(use_tc_tiling_on_sc=False),
)
def _gather_sc(table_hbm, idx_hbm, out_hbm, idx_v, rows_v, sem):
    wid = lax.axis_index("s") * NC + lax.axis_index("c")
    base = wid * B_PER_W
    for c in range(N_CH):
        off = base + c * CH
        pltpu.sync_copy(idx_hbm.at[pl.ds(off, CH)], idx_v)
        pltpu.async_copy(table_hbm.at[idx_v], rows_v, sem).wait()
        pltpu.sync_copy(rows_v, out_hbm.at[pl.ds(off, CH)])


def kernel(x, table):
    idx = x.reshape(-1).astype(jnp.int32)
    out = _gather_sc(table, idx)
    return out.reshape(x.shape + (D,))


# speedup vs baseline: 1.0164x; 1.0164x over previous
import functools

import jax
import jax.numpy as jnp
from jax import lax
from jax.experimental import pallas as pl
from jax.experimental.pallas import tpu as pltpu
from jax.experimental.pallas import tpu_sc as plsc

D = 16
NC, NS = 2, 16
NW = NC * NS
B_TOTAL = 4096 * 200
B_PER_W = B_TOTAL // NW
CH = 2560
N_CH = B_PER_W // CH

_mesh = plsc.VectorSubcoreMesh(core_axis_name="c", subcore_axis_name="s")


@functools.partial(
    pl.kernel,
    mesh=_mesh,
    out_type=jax.ShapeDtypeStruct((B_TOTAL, D), jnp.float32),
    scratch_types=[
        pltpu.VMEM((B_PER_W,), jnp.int32),
        pltpu.VMEM((CH, D), jnp.float32),
        pltpu.VMEM((CH, D), jnp.float32),
        pltpu.SemaphoreType.DMA,
        pltpu.SemaphoreType.DMA,
    ],
    compiler_params=pltpu.CompilerParams(use_tc_tiling_on_sc=False),
)
def _gather_sc(table_hbm, idx_hbm, out_hbm, idx_v, rows0, rows1, sem0, sem1):
    wid = lax.axis_index("s") * NC + lax.axis_index("c")
    base = wid * B_PER_W
    pltpu.sync_copy(idx_hbm.at[pl.ds(base, B_PER_W)], idx_v)
    bufs = (rows0, rows1)
    sems = (sem0, sem1)
    copies = [None, None]
    copies[0] = pltpu.async_copy(
        table_hbm.at[idx_v.at[pl.ds(0, CH)]], rows0, sem0)
    for c in range(N_CH):
        if c + 1 < N_CH:
            copies[(c + 1) % 2] = pltpu.async_copy(
                table_hbm.at[idx_v.at[pl.ds((c + 1) * CH, CH)]],
                bufs[(c + 1) % 2], sems[(c + 1) % 2])
        copies[c % 2].wait()
        pltpu.sync_copy(bufs[c % 2], out_hbm.at[pl.ds(base + c * CH, CH)])


def kernel(x, table):
    idx = x.reshape(-1).astype(jnp.int32)
    out = _gather_sc(table, idx)
    return out.reshape(x.shape + (D,))


# speedup vs baseline: 1.2544x; 1.2342x over previous
import functools

import jax
import jax.numpy as jnp
from jax import lax
from jax.experimental import pallas as pl
from jax.experimental.pallas import tpu as pltpu
from jax.experimental.pallas import tpu_sc as plsc

D = 16
NC, NS = 2, 16
NW = NC * NS
B, L = 4096, 200
NSB = (L // 8) * (B // 128) // NW
W_IDX = NSB * 8 * 128

_mesh = plsc.VectorSubcoreMesh(core_axis_name="c", subcore_axis_name="s")


@functools.partial(
    pl.kernel,
    mesh=_mesh,
    out_type=jax.ShapeDtypeStruct((L, 2, B // 128, 1024), jnp.float32),
    scratch_types=[
        pltpu.VMEM((W_IDX,), jnp.int32),
        pltpu.VMEM((128, D), jnp.float32),
        pltpu.VMEM((128, D), jnp.float32),
        pltpu.VMEM((2048,), jnp.float32),
        pltpu.SemaphoreType.DMA,
        pltpu.SemaphoreType.DMA,
    ],
    compiler_params=pltpu.CompilerParams(
        use_tc_tiling_on_sc=False, needs_layout_passes=False),
)
def _gather_sc(xv_hbm, table_hbm, out_hbm, idx_v, r0, r1, tb, sem0, sem1):
    wid = lax.axis_index("s") * NC + lax.axis_index("c")
    pltpu.sync_copy(xv_hbm.at[pl.ds(wid * W_IDX, W_IDX)], idx_v)
    jiota = lax.iota(jnp.int32, 16)
    scat_base = jiota * 128
    bufs = (r0, r1)
    sems = (sem0, sem1)

    def do_block(t, ll, copies):
        rows = bufs[ll % 2]
        copies[ll % 2].wait()

        def rowgrp(g, _):
            for u in range(8):
                k = g * 8 + u
                row = rows[k]
                plsc.store_scatter(tb, [scat_base + k], row)
            return 0

        lax.fori_loop(0, 16, rowgrp, 0)
        sbid = wid * NSB + t
        lh = sbid // 32
        bh = sbid % 32
        lidx = lh * 8 + ll
        pltpu.sync_copy(tb.at[pl.ds(0, 1024)], out_hbm.at[lidx, 0, bh])
        pltpu.sync_copy(tb.at[pl.ds(1024, 1024)], out_hbm.at[lidx, 1, bh])

    def outer(t, _):
        copies = [None, None]
        copies[0] = pltpu.async_copy(
            table_hbm.at[idx_v.at[pl.ds(t * 1024, 128)]], r0, sem0)
        for ll in range(8):
            if ll < 7:
                copies[(ll + 1) % 2] = pltpu.async_copy(
                    table_hbm.at[idx_v.at[pl.ds(t * 1024 + (ll + 1) * 128, 128)]],
                    bufs[(ll + 1) % 2], sems[(ll + 1) % 2])
            do_block(t, ll, copies)
        return 0

    lax.fori_loop(0, NSB, outer, 0)


def kernel(x, table):
    xv = (x.astype(jnp.int32).transpose(1, 0).reshape(25, 8, 32, 128)
          .transpose(0, 2, 1, 3).reshape(-1))
    o = _gather_sc(xv, table)
    return (o.reshape(L, 2, B // 128, 8, 128).transpose(2, 4, 0, 1, 3)
            .reshape(B, L, D))


# speedup vs baseline: 1.3489x; 1.0753x over previous
import functools

import jax
import jax.numpy as jnp
from jax import lax
from jax.experimental import pallas as pl
from jax.experimental.pallas import tpu as pltpu
from jax.experimental.pallas import tpu_sc as plsc

D = 16
NC, NS = 2, 16
NW = NC * NS
B, L = 4096, 200
NSB = L * (B // 1024) // NW
G = 1024

_mesh = plsc.VectorSubcoreMesh(core_axis_name="c", subcore_axis_name="s")


@functools.partial(
    pl.kernel,
    mesh=_mesh,
    out_type=jax.ShapeDtypeStruct((L, 2, (B // 128) * 1024), jnp.float32),
    scratch_types=[
        pltpu.VMEM((G,), jnp.int32),
        pltpu.VMEM((G,), jnp.int32),
        pltpu.VMEM((G, D), jnp.float32),
        pltpu.VMEM((G, D), jnp.float32),
        pltpu.VMEM((2 * 8192,), jnp.float32),
        pltpu.VMEM((2 * 8192,), jnp.float32),
        pltpu.SemaphoreType.DMA,
        pltpu.SemaphoreType.DMA,
        pltpu.SemaphoreType.DMA,
        pltpu.SemaphoreType.DMA,
        pltpu.SemaphoreType.DMA,
        pltpu.SemaphoreType.DMA,
    ],
    compiler_params=pltpu.CompilerParams(
        use_tc_tiling_on_sc=False, needs_layout_passes=False),
)
def _gather_sc(xv_hbm, table_hbm, out_hbm,
               i0, i1, r0, r1, t0, t1, si0, si1, sg0, sg1, sw0, sw1):
    wid = lax.axis_index("s") * NC + lax.axis_index("c")
    jiota = lax.iota(jnp.int32, 16)
    vj = (jiota // 8) * 8192 + (jiota % 8) * 128
    idxb = (i0, i1)
    rows = (r0, r1)
    tbs = (t0, t1)
    sis = (si0, si1)
    sgs = (sg0, sg1)
    sws = (sw0, sw1)

    def unit_coords(t):
        u = wid * NSB + t
        return u // 4, u % 4

    def issue_idx(t, p):
        l, bo = unit_coords(t)
        lh = l // 8
        ll = l % 8
        return [pltpu.async_copy(xv_hbm.at[lh, bo * 8 + i, ll],
                                 idxb[p].at[pl.ds(i * 128, 128)], sis[p])
                for i in range(8)]

    ic = {0: issue_idx(0, 0)}
    for c in ic[0]:
        c.wait()
    gc = {0: pltpu.async_copy(table_hbm.at[i0], r0, sg0)}
    if NSB > 1:
        ic[1] = issue_idx(1, 1)
    wc = {}
    for t in range(NSB):
        p = t % 2
        l, bo = unit_coords(t)
        gc[t].wait()
        if t + 1 < NSB:
            for c in ic[t + 1]:
                c.wait()
            q = (t + 1) % 2
            gc[t + 1] = pltpu.async_copy(table_hbm.at[idxb[q]], rows[q], sgs[q])
        if t + 2 < NSB:
            ic[t + 2] = issue_idx(t + 2, p)
        if t - 2 >= 0:
            for c in wc[t - 2]:
                c.wait()
        rr = rows[p]
        tb = tbs[p]

        def grp(g, _, rr=rr, tb=tb):
            sb = (g // 16) * 1024 + (g % 16) * 8
            for u8 in range(8):
                row = rr[g * 8 + u8]
                plsc.store_scatter(tb, [vj + (sb + u8)], row)
            return 0

        lax.fori_loop(0, G // 8, grp, 0)
        wc[t] = [
            pltpu.async_copy(tb.at[pl.ds(0, 8192)],
                             out_hbm.at[l, 0, pl.ds(bo * 8192, 8192)], sws[p]),
            pltpu.async_copy(tb.at[pl.ds(8192, 8192)],
                             out_hbm.at[l, 1, pl.ds(bo * 8192, 8192)], sws[p]),
        ]
    for t in range(max(0, NSB - 2), NSB):
        for c in wc[t]:
            c.wait()


def kernel(x, table):
    xv = (x.astype(jnp.int32).transpose(1, 0).reshape(25, 8, 32, 128)
          .transpose(0, 2, 1, 3))
    o = _gather_sc(xv, table)
    return (o.reshape(L, 2, B // 128, 8, 128).transpose(2, 4, 0, 1, 3)
            .reshape(B, L, D))
